# Initial kernel scaffold; baseline (speedup 1.0000x reference)
import jax
import jax.numpy as jnp
from jax.experimental import pallas as pl


def kernel(numerical_features, categorical_features, embedding_tables, Wb0, bb0, Wb1, bb1, Wb2, bb2, Wt0, bt0, Wt1, bt1, Wt2, bt2, Wt3, bt3, Wt4, bt4):
    raise NotImplementedError("write your pallas kernel here")


# speedup vs baseline: 2.2166x; 2.2166x over previous
import functools

import jax
import jax.numpy as jnp
from jax import lax
from jax.experimental import pallas as pl
from jax.experimental.pallas import tpu as pltpu
from jax.experimental.pallas import tpu_sc as plsc

B = 4096
NT = 26
V = 100000
D = 32
NF = NT + 1
R = B * NT
NW = 32
RPW = R // NW
CH = 128
NCH = RPW // CH
BBLK = 512
GRID = B // BBLK


def _sc_gather(table2d, idx2d):
    mesh = plsc.VectorSubcoreMesh(core_axis_name="c", subcore_axis_name="s")

    @functools.partial(
        pl.kernel,
        out_type=jax.ShapeDtypeStruct((R, D), jnp.float32),
        mesh=mesh,
        scratch_types=[
            pltpu.VMEM((NCH, CH), jnp.int32),
            pltpu.VMEM((RPW, D), jnp.float32),
            pltpu.SemaphoreType.DMA,
        ],
        compiler_params=pltpu.CompilerParams(use_tc_tiling_on_sc=False),
    )
    def k(table_hbm, idx_hbm, out_hbm, idx_v, rows_v, sem):
        wid = lax.axis_index("s") * 2 + lax.axis_index("c")
        pltpu.sync_copy(idx_hbm.at[wid], idx_v)
        cps = [
            pltpu.async_copy(
                table_hbm.at[idx_v.at[j]], rows_v.at[pl.ds(j * CH, CH)], sem
            )
            for j in range(NCH)
        ]
        for cp in cps:
            cp.wait()
        pltpu.sync_copy(rows_v, out_hbm.at[pl.ds(wid * RPW, RPW)])

    return k(table2d, idx2d)


def _relu(x):
    return jnp.maximum(x, 0.0)


def _dot(a, b):
    return jnp.dot(a, b, preferred_element_type=jnp.float32)


def _tc_body(numT, embT, wb0, b0, wb1, b1, wb2, b2,
             wt0, c0, wt1, c1, wt2, c2, wt3, c3, wt4, c4, out):
    x = _relu(_dot(wb0[...], numT[...]) + b0[...])
    x = _relu(_dot(wb1[...], x) + b1[...])
    x = _relu(_dot(wb2[...], x) + b2[...])

    e = embT[...]
    feats = [x] + [e[i * D:(i + 1) * D] for i in range(NT)]
    rows = []
    for i in range(1, NF):
        fi = feats[i]
        for j in range(i):
            rows.append(jnp.sum(fi * feats[j], axis=0, keepdims=True))
    zT = jnp.concatenate([x] + rows, axis=0)

    h = _relu(_dot(wt0[...], zT) + c0[...])
    h = _relu(_dot(wt1[...], h) + c1[...])
    h = _relu(_dot(wt2[...], h) + c2[...])
    h = _relu(_dot(wt3[...], h) + c3[...])
    out[...] = _dot(wt4[...], h) + c4[...]


def _full(shape):
    return pl.BlockSpec(shape, lambda g: (0, 0))


def _tc_forward(numT, embT, args):
    in_specs = [
        pl.BlockSpec((13, BBLK), lambda g: (0, g)),
        pl.BlockSpec((NT * D, BBLK), lambda g: (0, g)),
    ]
    for a in args:
        in_specs.append(_full(a.shape))
    return pl.pallas_call(
        _tc_body,
        grid=(GRID,),
        in_specs=in_specs,
        out_specs=pl.BlockSpec((1, BBLK), lambda g: (0, g)),
        out_shape=jax.ShapeDtypeStruct((1, B), jnp.float32),
    )(numT, embT, *args)


def kernel(numerical_features, categorical_features, embedding_tables,
           Wb0, bb0, Wb1, bb1, Wb2, bb2,
           Wt0, bt0, Wt1, bt1, Wt2, bt2, Wt3, bt3, Wt4, bt4):
    table2d = embedding_tables.reshape(NT * V, D)
    offs = (jnp.arange(NT, dtype=jnp.int32) * V)[None, :]
    idx2d = (categorical_features + offs).reshape(NW, NCH, CH)
    emb = _sc_gather(table2d, idx2d)

    embT = emb.reshape(B, NT * D).T
    numT = numerical_features.T
    args = [
        Wb0.T, bb0[:, None], Wb1.T, bb1[:, None], Wb2.T, bb2[:, None],
        Wt0.T, bt0[:, None], Wt1.T, bt1[:, None], Wt2.T, bt2[:, None],
        Wt3.T, bt3[:, None], Wt4.T, bt4[:, None],
    ]
    outT = _tc_forward(numT, embT, args)
    return outT.reshape(B, 1)


# speedup vs baseline: 12.1474x; 5.4802x over previous
import functools

import jax
import jax.numpy as jnp
from jax import lax
from jax.experimental import pallas as pl
from jax.experimental.pallas import tpu as pltpu
from jax.experimental.pallas import tpu_sc as plsc

B = 4096
NT = 26
V = 100000
D = 32
NF = NT + 1
R = B * NT
NW = 32
RPW = R // NW
CH = 128
NCH = RPW // CH
BBLK = 512
GRID = B // BBLK


RPW2 = (NT * D) // NW
NV16 = B // 16
VB = (V // CH) * CH
VT = V - VB


def _sc_scan_gather(tables_dmaj, catT, tails):
    mesh = plsc.VectorSubcoreMesh(core_axis_name="c", subcore_axis_name="s")

    @functools.partial(
        pl.kernel,
        out_type=jax.ShapeDtypeStruct((NT * D, B), jnp.float32),
        mesh=mesh,
        scratch_types=[
            pltpu.VMEM((B,), jnp.int32),
            pltpu.VMEM((V,), jnp.float32),
            pltpu.VMEM((B,), jnp.float32),
            pltpu.VMEM((8 * CH,), jnp.float32),
            pltpu.SemaphoreType.DMA,
        ],
    )
    def k(tab_hbm, idx_hbm, tails_hbm, out_hbm, idx_v, row_v, out_v, tail_v, sem):
        wid = lax.axis_index("s") * 2 + lax.axis_index("c")
        pltpu.sync_copy(tails_hbm.at[wid], tail_v)

        def body(kk, carry):
            t = kk
            d = wid
            r = t * D + d
            pltpu.sync_copy(idx_hbm.at[t], idx_v)
            pltpu.sync_copy(tab_hbm.at[t, d], row_v)
            toff = t * VT - VB

            @functools.partial(plsc.parallel_loop, 0, NV16, unroll=8)
            def _(m):
                iv = idx_v[pl.ds(m * 16, 16)]
                main = plsc.load_gather(row_v, [jnp.minimum(iv, VB - 1)])
                tpos = jnp.clip(iv + toff, 0, NT * VT - 1)
                tvals = plsc.load_gather(tail_v, [tpos])
                out_v[pl.ds(m * 16, 16)] = jnp.where(iv >= VB, tvals, main)

            pltpu.sync_copy(out_v, out_hbm.at[r])
            return carry

        lax.fori_loop(0, RPW2, body, 0)

    return k(tables_dmaj, catT, tails)


def _relu(x):
    return jnp.maximum(x, 0.0)


def _dot(a, b):
    return jnp.dot(a, b, preferred_element_type=jnp.float32)


def _tc_body(numT, embT, wb0, b0, wb1, b1, wb2, b2,
             wt0, c0, wt1, c1, wt2, c2, wt3, c3, wt4, c4, out):
    x = _relu(_dot(wb0[...], numT[...]) + b0[...])
    x = _relu(_dot(wb1[...], x) + b1[...])
    x = _relu(_dot(wb2[...], x) + b2[...])

    e = embT[...]
    feats = [x] + [e[i * D:(i + 1) * D] for i in range(NT)]
    rows = []
    for i in range(1, NF):
        fi = feats[i]
        for j in range(i):
            rows.append(jnp.sum(fi * feats[j], axis=0, keepdims=True))
    zT = jnp.concatenate([x] + rows, axis=0)

    h = _relu(_dot(wt0[...], zT) + c0[...])
    h = _relu(_dot(wt1[...], h) + c1[...])
    h = _relu(_dot(wt2[...], h) + c2[...])
    h = _relu(_dot(wt3[...], h) + c3[...])
    out[...] = _dot(wt4[...], h) + c4[...]


def _full(shape):
    return pl.BlockSpec(shape, lambda g: (0, 0))


def _tc_forward(numT, embT, args):
    in_specs = [
        pl.BlockSpec((13, BBLK), lambda g: (0, g)),
        pl.BlockSpec((NT * D, BBLK), lambda g: (0, g)),
    ]
    for a in args:
        in_specs.append(_full(a.shape))
    return pl.pallas_call(
        _tc_body,
        grid=(GRID,),
        in_specs=in_specs,
        out_specs=pl.BlockSpec((1, BBLK), lambda g: (0, g)),
        out_shape=jax.ShapeDtypeStruct((1, B), jnp.float32),
    )(numT, embT, *args)


def kernel(numerical_features, categorical_features, embedding_tables,
           Wb0, bb0, Wb1, bb1, Wb2, bb2,
           Wt0, bt0, Wt1, bt1, Wt2, bt2, Wt3, bt3, Wt4, bt4):
    tables_dmaj = embedding_tables.transpose(0, 2, 1)
    catT = categorical_features.T
    tails = jnp.pad(
        embedding_tables[:, VB:, :].transpose(2, 0, 1).reshape(D, NT * VT),
        ((0, 0), (0, 8 * CH - NT * VT)),
    )
    embT = _sc_scan_gather(tables_dmaj, catT, tails)
    numT = numerical_features.T
    args = [
        Wb0.T, bb0[:, None], Wb1.T, bb1[:, None], Wb2.T, bb2[:, None],
        Wt0.T, bt0[:, None], Wt1.T, bt1[:, None], Wt2.T, bt2[:, None],
        Wt3.T, bt3[:, None], Wt4.T, bt4[:, None],
    ]
    outT = _tc_forward(numT, embT, args)
    return outT.reshape(B, 1)
